# baseline (speedup 1.0000x reference)
import functools

import jax
import jax.numpy as jnp
from jax import lax
from jax.experimental import pallas as pl
from jax.experimental.pallas import tpu as pltpu
from jax.experimental.pallas import tpu_sc as plsc

K = 16
GB = 128
BLK = 2048
NEG = float("-inf")


def _tc_body(nblk, n_items, state_ref, emb_ref,
             lout, gmax_out, rmax_out, rsum_out, rmax, rsum):
    j = pl.program_id(0)
    ngr = BLK // GB
    b = state_ref.shape[0]

    @pl.when(j == 0)
    def _init():
        rmax[...] = jnp.full_like(rmax, NEG)
        rsum[...] = jnp.zeros_like(rsum)

    l = lax.dot_general(
        state_ref[...], emb_ref[...],
        dimension_numbers=(((1,), (1,)), ((), ())),
        preferred_element_type=jnp.float32)
    col = lax.broadcasted_iota(jnp.int32, l.shape, 1) + j * BLK
    l = jnp.where(col < n_items, l, NEG)

    for g in range(ngr):
        lout[:, g, :, :] = l[:, g * GB:(g + 1) * GB].reshape(b // 8, 8, GB)

    gmax = jnp.concatenate(
        [jnp.max(l[:, g * GB:(g + 1) * GB], axis=1, keepdims=True)
         for g in range(ngr)], axis=1)
    for s in range(8):
        @pl.when(j % 8 == s)
        def _slot(s=s):
            gmax_out[0, :, s * ngr:(s + 1) * ngr] = gmax

    bm = jnp.max(gmax, axis=1, keepdims=True)
    new_max = jnp.maximum(rmax[...], bm)
    rsum[...] = (rsum[...] * jnp.exp(rmax[...] - new_max)
                 + jnp.sum(jnp.exp(l - new_max), axis=1, keepdims=True))
    rmax[...] = new_max

    @pl.when(j == nblk - 1)
    def _fin():
        rmax_out[...] = jnp.broadcast_to(rmax[...], (b, GB))
        rsum_out[...] = jnp.broadcast_to(rsum[...], (b, GB))


def _tc_stage(state, emb):
    b, d = state.shape
    n = emb.shape[0]
    npad = -(-n // BLK) * BLK
    if npad != n:
        emb = jnp.pad(emb, ((0, npad - n), (0, 0)))
    nblk = npad // BLK
    ng = npad // GB
    ngr = BLK // GB

    grid_spec = pltpu.PrefetchScalarGridSpec(
        num_scalar_prefetch=0,
        grid=(nblk,),
        in_specs=[
            pl.BlockSpec((b, d), lambda j: (0, 0)),
            pl.BlockSpec((BLK, d), lambda j: (j, 0)),
        ],
        out_specs=[
            pl.BlockSpec((b // 8, ngr, 8, GB), lambda j: (0, j, 0, 0)),
            pl.BlockSpec((1, b, GB), lambda j: (j // 8, 0, 0)),
            pl.BlockSpec((b, GB), lambda j: (0, 0)),
            pl.BlockSpec((b, GB), lambda j: (0, 0)),
        ],
        scratch_shapes=[
            pltpu.VMEM((b, 1), jnp.float32),
            pltpu.VMEM((b, 1), jnp.float32),
        ],
    )
    return pl.pallas_call(
        functools.partial(_tc_body, nblk, n),
        grid_spec=grid_spec,
        out_shape=[
            jax.ShapeDtypeStruct((b // 8, ng, 8, GB), jnp.float32),
            jax.ShapeDtypeStruct((-(-nblk // 8), b, GB), jnp.float32),
            jax.ShapeDtypeStruct((b, GB), jnp.float32),
            jax.ShapeDtypeStruct((b, GB), jnp.float32),
        ],
    )(state, emb)


def _bcast(vec, lane):
    idx = jnp.zeros((16,), jnp.int32) + lane
    return lax.gather(
        vec, idx[:, None],
        dimension_numbers=lax.GatherDimensionNumbers(
            offset_dims=(), collapsed_slice_dims=(0,),
            start_index_map=(0,)),
        slice_sizes=(1,),
        mode=lax.GatherScatterMode.PROMISE_IN_BOUNDS)


def _sc_stage(table, gmax3, rmaxf, rsumf, b, ng, nblk, n_items):
    bk = b * K
    info = plsc.get_sparse_core_info()
    nw = info.num_cores * info.num_subcores
    rw = b // nw
    nseg = rw * K
    vpr = GB // 16
    nchain = 8
    nwin = -(-nblk // 8)

    mesh = plsc.VectorSubcoreMesh(core_axis_name="c", subcore_axis_name="s")

    @functools.partial(
        pl.kernel, mesh=mesh,
        compiler_params=pltpu.CompilerParams(needs_layout_passes=False),
        out_type=[
            jax.ShapeDtypeStruct((b, GB), jnp.int32),
            jax.ShapeDtypeStruct((b, GB), jnp.float32),
        ],
        scratch_types=[
            pltpu.VMEM((nseg,), jnp.int32),
            pltpu.VMEM((nseg // GB, GB), jnp.int32),
            pltpu.VMEM((nseg, GB), jnp.float32),
            pltpu.VMEM((nwin, rw, GB), jnp.float32),
            pltpu.VMEM((rw, GB), jnp.float32),
            pltpu.VMEM((rw, GB), jnp.float32),
            pltpu.VMEM((rw, GB), jnp.int32),
            pltpu.VMEM((rw, GB), jnp.float32),
            pltpu.SemaphoreType.DMA,
        ],
    )
    def sck(table_h, gmax_h, rmax_h, rsum_h, oidx_h, oprob_h,
            gids_v, tidx_v, gath_v, gmv, rmax_v, rsum_v, obi_v, obp_v, sem):
        cid = lax.axis_index("c")
        sid = lax.axis_index("s")
        wid = sid * info.num_cores + cid
        base = wid * rw

        pltpu.sync_copy(gmax_h.at[:, pl.ds(base, rw), :], gmv)
        pltpu.sync_copy(rmax_h.at[pl.ds(base, rw), :], rmax_v)
        pltpu.sync_copy(rsum_h.at[pl.ds(base, rw), :], rsum_v)

        iota = lax.broadcasted_iota(jnp.int32, (16,), 0)

        def merge2(a, bch):
            av, ai = a
            bv, bi = bch
            bvr = lax.rev(bv, (0,))
            bir = lax.rev(bi, (0,))
            keep_a = (av > bvr) | ((av == bvr) & (ai < bir))
            nv = jnp.maximum(av, bvr)
            ni = jnp.where(keep_a, ai, bir)
            return plsc.sort_key_val(nv, ni, descending=False)

        def grp_body(r, carry):
            nch1 = 4
            chains = [(jnp.full((16,), NEG, jnp.float32),
                       jnp.zeros((16,), jnp.int32)) for _ in range(nch1)]
            for k in range(-(-nwin * 8 // nch1)):
                for ch in range(nch1):
                    u = k * nch1 + ch
                    if u >= nwin * 8:
                        continue
                    t, q = u // 8, u % 8
                    tv, ti = chains[ch]
                    v = gmv[t, r, pl.ds(q * 16, 16)]
                    vi = iota + u * 16
                    v = jnp.where(vi < ng, v, NEG)
                    vd, vid = plsc.sort_key_val(v, vi, descending=True)
                    keep_t = (tv > vd) | ((tv == vd) & (ti < vid))
                    nv = jnp.maximum(tv, vd)
                    ni = jnp.where(keep_t, ti, vid)
                    chains[ch] = plsc.sort_key_val(nv, ni, descending=False)
            m01 = merge2(chains[0], chains[1])
            m23 = merge2(chains[2], chains[3])
            _, gii = merge2(m01, m23)
            gii = lax.rev(gii, (0,))
            gids_v[pl.ds(r * K, 16)] = gii
            rr = base + r
            tidx_v[r // 8, pl.ds((r % 8) * 16, 16)] = (
                gii * 8 + (rr // 8) * (ng * 8) + (rr % 8))
            return carry

        lax.fori_loop(0, rw, grp_body, 0)

        cps = []
        for c in range(nseg // GB):
            cps.append(pltpu.async_copy(
                table_h.at[tidx_v.at[c]],
                gath_v.at[pl.ds(c * GB, GB)], sem))
        for cp in cps:
            cp.wait()

        def row_body(r, carry):
            gv_row = gids_v[pl.ds(r * K, 16)]
            chains = [(jnp.full((16,), NEG, jnp.float32),
                       jnp.zeros((16,), jnp.int32)) for _ in range(nchain)]
            vib = [_bcast(gv_row, s) * GB + iota for s in range(K)]

            def do_merge(args):
                tv, ti, v, vi = args
                vd, vid = plsc.sort_key_val(v, vi, descending=True)
                keep_t = (tv > vd) | ((tv == vd) & (ti < vid))
                nv = jnp.maximum(tv, vd)
                ni = jnp.where(keep_t, ti, vid)
                rv, ri = plsc.sort_key_val(nv, ni, descending=False)
                return rv, ri

            def no_merge(args):
                tv, ti, _, _ = args
                return tv, ti

            for k in range(K * vpr // nchain):
                for ch in range(nchain):
                    s = ch + (k // vpr) * nchain
                    w = k % vpr
                    tv, ti = chains[ch]
                    v = gath_v[r * K + s, pl.ds(w * 16, 16)]
                    vi = vib[s] + w * 16
                    v = jnp.where(vi < n_items, v, NEG)
                    hit = jnp.max(v) >= jnp.min(tv)
                    chains[ch] = lax.cond(
                        hit, do_merge, no_merge, (tv, ti, v, vi))
            while len(chains) > 1:
                chains = [merge2(chains[i], chains[i + 1])
                          for i in range(0, len(chains), 2)]
            tv, ti = chains[0]

            rm = rmax_v[r, pl.ds(0, 16)]
            rs = rsum_v[r, pl.ds(0, 16)]
            pv = jnp.exp(tv - rm) / rs

            rankv = jnp.zeros((16,), jnp.int32)
            for t in range(K):
                pt = _bcast(pv, t)
                it = _bcast(ti, t)
                m = (pv > pt) | ((pv == pt) & (ti < it))
                cnt = plsc.all_reduce_population_count(m)
                rankv = jnp.where(iota == t, cnt, rankv)
            _, oi = plsc.sort_key_val(rankv, ti, descending=False)
            _, op = plsc.sort_key_val(rankv, pv, descending=False)
            obi_v[r, pl.ds(0, 16)] = oi
            obp_v[r, pl.ds(0, 16)] = op
            return carry

        lax.fori_loop(0, rw, row_body, 0)

        pltpu.sync_copy(obi_v, oidx_h.at[pl.ds(base, rw), :])
        pltpu.sync_copy(obp_v, oprob_h.at[pl.ds(base, rw), :])

    return sck(table, gmax3, rmaxf, rsumf)


@jax.jit
def kernel(state, item_embeddings):
    b = state.shape[0]
    n = item_embeddings.shape[0]
    lout, gmax3, rmax, rsum = _tc_stage(state, item_embeddings)
    ng = lout.shape[1]
    nblk = -(-n // BLK)
    table = lout.reshape(b * ng, GB)
    oidx, oprob = _sc_stage(table, gmax3, rmax, rsum, b, ng, nblk, n)
    return oidx[:, :K], oprob[:, :K]


# speedup vs baseline: 1.2811x; 1.2811x over previous
import functools

import jax
import jax.numpy as jnp
from jax import lax
from jax.experimental import pallas as pl
from jax.experimental.pallas import tpu as pltpu
from jax.experimental.pallas import tpu_sc as plsc

K = 16
GB = 128
BLK = 2048
NEG = float("-inf")


def _tc_body(nblk, n_items, state_ref, emb_ref,
             lout, gmax_out, rmax_out, rsum_out, rmax, rsum):
    j = pl.program_id(0)
    ngr = BLK // GB
    b = state_ref.shape[1]

    @pl.when(j == 0)
    def _init():
        rmax[...] = jnp.full_like(rmax, NEG)
        rsum[...] = jnp.zeros_like(rsum)

    l = lax.dot_general(
        state_ref[...], emb_ref[...],
        dimension_numbers=(((0,), (0,)), ((), ())),
        preferred_element_type=jnp.float32)
    col = lax.broadcasted_iota(jnp.int32, l.shape, 1) + j * BLK
    l = jnp.where(col < n_items, l, NEG)

    for g in range(ngr):
        lout[:, g, :, :] = l[:, g * GB:(g + 1) * GB].reshape(b // 8, 8, GB)

    gmax = jnp.concatenate(
        [jnp.max(l[:, g * GB:(g + 1) * GB], axis=1, keepdims=True)
         for g in range(ngr)], axis=1)
    for s in range(8):
        @pl.when(j % 8 == s)
        def _slot(s=s):
            gmax_out[0, :, s * ngr:(s + 1) * ngr] = gmax

    bm = jnp.max(gmax, axis=1, keepdims=True)
    new_max = jnp.maximum(rmax[...], bm)
    rsum[...] = (rsum[...] * jnp.exp(rmax[...] - new_max)
                 + jnp.sum(jnp.exp(l - new_max), axis=1, keepdims=True))
    rmax[...] = new_max

    @pl.when(j == nblk - 1)
    def _fin():
        rmax_out[...] = jnp.broadcast_to(rmax[...], (b, GB))
        rsum_out[...] = jnp.broadcast_to(rsum[...], (b, GB))


def _tc_stage(state_t, emb_t):
    d, b = state_t.shape
    n = emb_t.shape[1]
    nblk = -(-n // BLK)
    ng = nblk * (BLK // GB)
    ngr = BLK // GB

    grid_spec = pltpu.PrefetchScalarGridSpec(
        num_scalar_prefetch=0,
        grid=(nblk,),
        in_specs=[
            pl.BlockSpec((d, b), lambda j: (0, 0)),
            pl.BlockSpec((d, BLK), lambda j: (0, j)),
        ],
        out_specs=[
            pl.BlockSpec((b // 8, ngr, 8, GB), lambda j: (0, j, 0, 0)),
            pl.BlockSpec((1, b, GB), lambda j: (j // 8, 0, 0)),
            pl.BlockSpec((b, GB), lambda j: (0, 0)),
            pl.BlockSpec((b, GB), lambda j: (0, 0)),
        ],
        scratch_shapes=[
            pltpu.VMEM((b, 1), jnp.float32),
            pltpu.VMEM((b, 1), jnp.float32),
        ],
    )
    return pl.pallas_call(
        functools.partial(_tc_body, nblk, n),
        grid_spec=grid_spec,
        out_shape=[
            jax.ShapeDtypeStruct((b // 8, ng, 8, GB), jnp.float32),
            jax.ShapeDtypeStruct((-(-nblk // 8), b, GB), jnp.float32),
            jax.ShapeDtypeStruct((b, GB), jnp.float32),
            jax.ShapeDtypeStruct((b, GB), jnp.float32),
        ],
    )(state_t, emb_t)


def _bcast(vec, lane):
    idx = jnp.zeros((16,), jnp.int32) + lane
    return lax.gather(
        vec, idx[:, None],
        dimension_numbers=lax.GatherDimensionNumbers(
            offset_dims=(), collapsed_slice_dims=(0,),
            start_index_map=(0,)),
        slice_sizes=(1,),
        mode=lax.GatherScatterMode.PROMISE_IN_BOUNDS)


def _sc_stage(table, gmax3, rmaxf, rsumf, b, ng, nblk, n_items):
    bk = b * K
    info = plsc.get_sparse_core_info()
    nw = info.num_cores * info.num_subcores
    rw = b // nw
    nseg = rw * K
    vpr = GB // 16
    nchain = 8
    nwin = -(-nblk // 8)

    mesh = plsc.VectorSubcoreMesh(core_axis_name="c", subcore_axis_name="s")

    @functools.partial(
        pl.kernel, mesh=mesh,
        compiler_params=pltpu.CompilerParams(needs_layout_passes=False),
        out_type=[
            jax.ShapeDtypeStruct((b, GB), jnp.int32),
            jax.ShapeDtypeStruct((b, GB), jnp.float32),
        ],
        scratch_types=[
            pltpu.VMEM((nseg,), jnp.int32),
            pltpu.VMEM((nseg // GB, GB), jnp.int32),
            pltpu.VMEM((nseg, GB), jnp.float32),
            pltpu.VMEM((nwin, rw, GB), jnp.float32),
            pltpu.VMEM((rw, GB), jnp.float32),
            pltpu.VMEM((rw, GB), jnp.float32),
            pltpu.VMEM((rw, GB), jnp.int32),
            pltpu.VMEM((rw, GB), jnp.float32),
            pltpu.SemaphoreType.DMA,
        ],
    )
    def sck(table_h, gmax_h, rmax_h, rsum_h, oidx_h, oprob_h,
            gids_v, tidx_v, gath_v, gmv, rmax_v, rsum_v, obi_v, obp_v, sem):
        cid = lax.axis_index("c")
        sid = lax.axis_index("s")
        wid = sid * info.num_cores + cid
        base = wid * rw

        pltpu.sync_copy(gmax_h.at[:, pl.ds(base, rw), :], gmv)
        pltpu.sync_copy(rmax_h.at[pl.ds(base, rw), :], rmax_v)
        pltpu.sync_copy(rsum_h.at[pl.ds(base, rw), :], rsum_v)

        iota = lax.broadcasted_iota(jnp.int32, (16,), 0)

        def merge2(a, bch):
            av, ai = a
            bv, bi = bch
            bvr = lax.rev(bv, (0,))
            bir = lax.rev(bi, (0,))
            keep_a = (av > bvr) | ((av == bvr) & (ai < bir))
            nv = jnp.maximum(av, bvr)
            ni = jnp.where(keep_a, ai, bir)
            return plsc.sort_key_val(nv, ni, descending=False)

        def grp_body(r, carry):
            nch1 = 4
            chains = [(jnp.full((16,), NEG, jnp.float32),
                       jnp.zeros((16,), jnp.int32)) for _ in range(nch1)]
            for k in range(-(-nwin * 8 // nch1)):
                for ch in range(nch1):
                    u = k * nch1 + ch
                    if u >= nwin * 8:
                        continue
                    t, q = u // 8, u % 8
                    tv, ti = chains[ch]
                    v = gmv[t, r, pl.ds(q * 16, 16)]
                    vi = iota + u * 16
                    v = jnp.where(vi < ng, v, NEG)
                    vd, vid = plsc.sort_key_val(v, vi, descending=True)
                    keep_t = (tv > vd) | ((tv == vd) & (ti < vid))
                    nv = jnp.maximum(tv, vd)
                    ni = jnp.where(keep_t, ti, vid)
                    chains[ch] = plsc.sort_key_val(nv, ni, descending=False)
            m01 = merge2(chains[0], chains[1])
            m23 = merge2(chains[2], chains[3])
            _, gii = merge2(m01, m23)
            gids_v[pl.ds(r * K, 16)] = gii
            rr = base + r
            tidx_v[r // 8, pl.ds((r % 8) * 16, 16)] = (
                gii * 8 + (rr // 8) * (ng * 8) + (rr % 8))
            return carry

        lax.fori_loop(0, rw, grp_body, 0)

        cps = []
        for c in range(nseg // GB):
            cps.append(pltpu.async_copy(
                table_h.at[tidx_v.at[c]],
                gath_v.at[pl.ds(c * GB, GB)], sem))
        for cp in cps:
            cp.wait()

        def row_body(r, carry):
            gv_row = gids_v[pl.ds(r * K, 16)]
            chains = [(jnp.full((16,), NEG, jnp.float32),
                       jnp.zeros((16,), jnp.int32)) for _ in range(nchain)]
            vib = [_bcast(gv_row, s) * GB + iota for s in range(K)]
            for k in range(K * vpr // nchain):
                for ch in range(nchain):
                    s = ch + (k // vpr) * nchain
                    w = k % vpr
                    tv, ti = chains[ch]
                    v = gath_v[r * K + s, pl.ds(w * 16, 16)]
                    vi = vib[s] + w * 16
                    v = jnp.where(vi < n_items, v, NEG)
                    vd, vid = plsc.sort_key_val(v, vi, descending=True)
                    keep_t = (tv > vd) | ((tv == vd) & (ti < vid))
                    nv = jnp.maximum(tv, vd)
                    ni = jnp.where(keep_t, ti, vid)
                    chains[ch] = plsc.sort_key_val(nv, ni, descending=False)
            while len(chains) > 1:
                chains = [merge2(chains[i], chains[i + 1])
                          for i in range(0, len(chains), 2)]
            tv, ti = chains[0]

            rm = rmax_v[r, pl.ds(0, 16)]
            rs = rsum_v[r, pl.ds(0, 16)]
            pv = jnp.exp(tv - rm) / rs

            rankv = jnp.zeros((16,), jnp.int32)
            for t in range(K):
                pt = _bcast(pv, t)
                it = _bcast(ti, t)
                m = (pv > pt) | ((pv == pt) & (ti < it))
                cnt = plsc.all_reduce_population_count(m)
                rankv = jnp.where(iota == t, cnt, rankv)
            _, oi = plsc.sort_key_val(rankv, ti, descending=False)
            _, op = plsc.sort_key_val(rankv, pv, descending=False)
            obi_v[r, pl.ds(0, 16)] = oi
            obp_v[r, pl.ds(0, 16)] = op
            return carry

        lax.fori_loop(0, rw, row_body, 0)

        pltpu.sync_copy(obi_v, oidx_h.at[pl.ds(base, rw), :])
        pltpu.sync_copy(obp_v, oprob_h.at[pl.ds(base, rw), :])

    return sck(table, gmax3, rmaxf, rsumf)


@jax.jit
def kernel(state, item_embeddings):
    b = state.shape[0]
    n = item_embeddings.shape[0]
    lout, gmax3, rmax, rsum = _tc_stage(state.T, item_embeddings.T)
    ng = lout.shape[1]
    nblk = -(-n // BLK)
    table = lout.reshape(b * ng, GB)
    oidx, oprob = _sc_stage(table, gmax3, rmax, rsum, b, ng, nblk, n)
    return oidx[:, :K], oprob[:, :K]
